# Initial kernel scaffold; baseline (speedup 1.0000x reference)
import jax
import jax.numpy as jnp
from jax.experimental import pallas as pl


def kernel(y):
    raise NotImplementedError("write your pallas kernel here")


# speedup vs baseline: 7.1950x; 7.1950x over previous
import functools

import jax
import jax.numpy as jnp
import numpy as np
from jax import lax
from jax.experimental import pallas as pl
from jax.experimental.pallas import tpu as pltpu
from jax.experimental.pallas import tpu_sc as plsc

B, C, H, W = 16, 768, 32, 32
HW = H * W
NROWS = B * C
RANGES = (0.1, 0.3)

NC, NS = 2, 16
NWORK = NC * NS

CH = 40
ZCH = 24


def _static_plan():
    rng = np.random.RandomState(0)
    mix = []
    for i in range(0, B - 1, 2):
        frac = rng.rand() * (RANGES[1] - RANGES[0]) + RANGES[0]
        num_first = int(C * frac)
        perm = rng.permutation(C)
        mix.append((i, perm[:num_first].copy()))
    num_first = int(C * (rng.rand() * (RANGES[1] - RANGES[0]) + RANGES[0]))
    num_second = int(C * (rng.rand() * (RANGES[1] - RANGES[0]) + RANGES[0]))
    perm = rng.permutation(C)
    drop = perm[num_first:num_first + num_second].copy()

    src_b = np.tile(np.arange(B, dtype=np.int64)[:, None], (1, C))
    for i, chans in mix:
        src_b[i, chans] = i + 1
    keep = np.ones((B, C), dtype=bool)
    keep[:, drop] = False

    rows = np.arange(NROWS, dtype=np.int64).reshape(B, C)
    src_row = src_b * C + np.arange(C, dtype=np.int64)[None, :]
    gdst = rows[keep]
    gsrc = src_row[keep]
    zdst = rows[~keep]
    return gsrc.astype(np.int32), gdst.astype(np.int32), zdst.astype(np.int32)


def _pad_to(a, n):
    return np.concatenate([a, np.full(n - a.size, a[-1], a.dtype)]) if a.size < n else a


def _build_index_tables():
    gsrc, gdst, zdst = _static_plan()
    nch = max(1, -(-(-(-gsrc.size // NWORK)) // CH))
    per_w = nch * CH
    gsrc = _pad_to(gsrc, NWORK * per_w).reshape(NWORK, nch, CH)
    gdst = _pad_to(gdst, NWORK * per_w).reshape(NWORK, nch, CH)
    nzch = max(1, -(-(-(-zdst.size // NWORK)) // ZCH))
    zper_w = nzch * ZCH
    zdst = _pad_to(zdst, NWORK * zper_w).reshape(NWORK, nzch, ZCH)
    return gsrc, gdst, zdst, nch, nzch


_GSRC, _GDST, _ZDST, _NCH, _NZCH = _build_index_tables()


def _body(ytab, gsrc_h, gdst_h, zdst_h, zeros_h, out,
          gsrc_v, gdst_v, zdst_v, zbuf, buf0, buf1,
          sg0, sg1, ss0, ss1, sz):
    w = lax.axis_index("s") * NC + lax.axis_index("c")

    pltpu.sync_copy(gsrc_h.at[w], gsrc_v)
    pltpu.sync_copy(gdst_h.at[w], gdst_v)
    pltpu.sync_copy(zdst_h.at[w], zdst_v)
    pltpu.sync_copy(zeros_h, zbuf)

    zcps = [pltpu.async_copy(zbuf, out.at[zdst_v.at[j]], sz)
            for j in range(_NZCH)]

    bufs = (buf0, buf1)
    sgs = (sg0, sg1)
    sss = (ss0, ss1)

    def gather(j, b):
        return pltpu.async_copy(ytab.at[gsrc_v.at[j]], bufs[b], sgs[b])

    def scatter(j, b):
        return pltpu.async_copy(bufs[b], out.at[gdst_v.at[j]], sss[b])

    gcps = [None, None]
    scps = [None, None]
    gcps[0] = gather(0, 0)
    if _NCH > 1:
        gcps[1] = gather(1, 1)
    for j in range(_NCH):
        b = j % 2
        gcps[b].wait()
        scps[b] = scatter(j, b)
        if j + 2 < _NCH:
            scps[b].wait()
            gcps[b] = gather(j + 2, b)
    for cp in scps:
        if cp is not None:
            cp.wait()
    for cp in zcps:
        cp.wait()


def kernel(y):
    ytab = y.reshape(NROWS, HW)
    mesh = plsc.VectorSubcoreMesh(core_axis_name="c", subcore_axis_name="s",
                                  num_cores=NC, num_subcores=NS)
    run = pl.kernel(
        _body,
        out_type=jax.ShapeDtypeStruct((NROWS, HW), jnp.float32),
        mesh=mesh,
        scratch_types=[
            pltpu.VMEM((_NCH, CH), jnp.int32),
            pltpu.VMEM((_NCH, CH), jnp.int32),
            pltpu.VMEM((_NZCH, ZCH), jnp.int32),
            pltpu.VMEM((ZCH, HW), jnp.float32),
            pltpu.VMEM((CH, HW), jnp.float32),
            pltpu.VMEM((CH, HW), jnp.float32),
            pltpu.SemaphoreType.DMA,
            pltpu.SemaphoreType.DMA,
            pltpu.SemaphoreType.DMA,
            pltpu.SemaphoreType.DMA,
            pltpu.SemaphoreType.DMA,
        ],
    )
    out = run(ytab,
              jnp.asarray(_GSRC), jnp.asarray(_GDST), jnp.asarray(_ZDST),
              jnp.zeros((ZCH, HW), jnp.float32))
    return out.reshape(B, C, H, W)
